# Initial kernel scaffold; baseline (speedup 1.0000x reference)
import jax
import jax.numpy as jnp
from jax.experimental import pallas as pl


def kernel(x, W):
    raise NotImplementedError("write your pallas kernel here")


# speedup vs baseline: 1.0618x; 1.0618x over previous
import jax
import jax.numpy as jnp
from jax import lax
from jax.experimental import pallas as pl
from jax.experimental.pallas import tpu as pltpu
from jax.experimental.pallas import tpu_sc as plsc

_K = 8192
_D = 64
_N = 8192
_BN = 512
_BK = 4096
_NBLK = _N // _BN


def _dist_argmin_body(z_ref, z2_ref, w2d_ref, w2_ref, idx_ref, dsum_ref):
    z = z_ref[...]
    z2 = z2_ref[...]
    best_val = jnp.full((_BN, 1), jnp.inf, jnp.float32)
    best_idx = jnp.zeros((_BN, 1), jnp.int32)
    dmin = jnp.full((_BN, 1), jnp.inf, jnp.float32)
    for kc in range(_K // _BK):
        wblk = w2d_ref[kc * _BK:(kc + 1) * _BK, :]
        w2blk = w2_ref[:, kc * _BK:(kc + 1) * _BK]
        mm2 = lax.dot_general(z, wblk, (((1,), (1,)), ((), ())),
                              preferred_element_type=jnp.float32)
        d = (z2 + w2blk) - mm2
        bmin = jnp.min(d, axis=1, keepdims=True)
        iot = lax.broadcasted_iota(jnp.int32, (_BN, _BK), 1)
        bidx = jnp.min(jnp.where(d == bmin, iot, _K),
                       axis=1, keepdims=True) + kc * _BK
        upd = bmin < best_val
        bmin16 = bmin.astype(jnp.bfloat16).astype(jnp.float32)
        best_val = jnp.where(upd, bmin16, best_val)
        best_idx = jnp.where(upd, bidx, best_idx)
        dmin = jnp.minimum(dmin, bmin)
    idx_ref[...] = best_idx
    dsum_ref[...] = jnp.full((1, 8, 128), jnp.sum(dmin), jnp.float32)


_GC = 128


def _gather_body(w_hbm, idx_hbm, out_hbm, idx_v, rows_v, sem):
    info = plsc.get_sparse_core_info()
    nw = info.num_cores * info.num_subcores
    wid = lax.axis_index("s") * info.num_cores + lax.axis_index("c")
    bpw = _N // nw
    for cc in range(bpw // _GC):
        base = wid * bpw + cc * _GC
        pltpu.sync_copy(idx_hbm.at[pl.ds(base, _GC)], idx_v)
        pltpu.async_copy(w_hbm.at[idx_v], rows_v, sem).wait()
        pltpu.sync_copy(rows_v, out_hbm.at[pl.ds(base, _GC)])


def kernel(x, W):
    x_p = jnp.transpose(x, (0, 2, 3, 1))
    z_flat = x_p.reshape(-1, _D)
    z2 = jnp.sum(z_flat ** 2, axis=1, keepdims=True)
    w2 = jnp.sum(W ** 2, axis=1).reshape(1, _K)
    w2d = W + W

    idx2d, dsum = pl.pallas_call(
        _dist_argmin_body,
        grid=(_NBLK,),
        in_specs=[
            pl.BlockSpec((_BN, _D), lambda i: (i, 0)),
            pl.BlockSpec((_BN, 1), lambda i: (i, 0)),
            pl.BlockSpec((_K, _D), lambda i: (0, 0)),
            pl.BlockSpec((1, _K), lambda i: (0, 0)),
        ],
        out_specs=[
            pl.BlockSpec((_BN, 1), lambda i: (i, 0)),
            pl.BlockSpec((1, 8, 128), lambda i: (i, 0, 0)),
        ],
        out_shape=[
            jax.ShapeDtypeStruct((_N, 1), jnp.int32),
            jax.ShapeDtypeStruct((_NBLK, 8, 128), jnp.float32),
        ],
        compiler_params=pltpu.CompilerParams(
            dimension_semantics=("parallel",)),
    )(z_flat, z2, w2d, w2)

    encoding_index = idx2d.reshape(_N)

    w_pad = jnp.concatenate([W, jnp.zeros((_K, 128 - _D), jnp.float32)], axis=1)
    gather = pl.kernel(
        _gather_body,
        mesh=plsc.VectorSubcoreMesh(core_axis_name="c", subcore_axis_name="s"),
        out_type=jax.ShapeDtypeStruct((_N, 128), jnp.float32),
        scratch_types=[
            pltpu.VMEM((_GC,), jnp.int32),
            pltpu.VMEM((_GC, 128), jnp.float32),
            pltpu.SemaphoreType.DMA,
        ],
    )
    embed_flat = gather(w_pad, encoding_index)[:, :_D]

    loss = 1.25 * jnp.sum(dsum[:, 0, 0]) / (_N * _D)
    embed_out = jnp.transpose(embed_flat.reshape(8, 32, 32, _D), (0, 3, 1, 2))
    return (embed_out, loss, encoding_index)


# speedup vs baseline: 1.1154x; 1.0505x over previous
import jax
import jax.numpy as jnp
from jax import lax
from jax.experimental import pallas as pl
from jax.experimental.pallas import tpu as pltpu
from jax.experimental.pallas import tpu_sc as plsc

_K = 8192
_D = 64
_N = 8192
_BN = 512
_BK = 4096
_NBLK = _N // _BN


def _dist_argmin_body(zt_ref, z2_ref, w2d_ref, w2_ref, idx_ref, dsum_ref):
    zt = zt_ref[0]
    z2 = z2_ref[...]
    best_val = jnp.full((1, _BN), jnp.inf, jnp.float32)
    best_idx = jnp.zeros((1, _BN), jnp.int32)
    dmin = jnp.full((1, _BN), jnp.inf, jnp.float32)
    for kc in range(_K // _BK):
        wblk = w2d_ref[kc * _BK:(kc + 1) * _BK, :]
        w2blk = w2_ref[kc * _BK:(kc + 1) * _BK, :]
        mm2 = lax.dot_general(wblk, zt, (((1,), (0,)), ((), ())),
                              preferred_element_type=jnp.float32)
        d = (z2 + w2blk) - mm2
        bmin = jnp.min(d, axis=0, keepdims=True)
        iot = lax.broadcasted_iota(jnp.int32, (_BK, _BN), 0)
        bidx = jnp.min(jnp.where(d == bmin, iot, _K),
                       axis=0, keepdims=True) + kc * _BK
        upd = bmin < best_val
        bmin16 = bmin.astype(jnp.bfloat16).astype(jnp.float32)
        best_val = jnp.where(upd, bmin16, best_val)
        best_idx = jnp.where(upd, bidx, best_idx)
        dmin = jnp.minimum(dmin, bmin)
    idx_ref[...] = best_idx.reshape(1, 1, _BN)
    dsum_ref[...] = jnp.full((1, 8, 128), jnp.sum(dmin), jnp.float32)


_GC = 128


def _gather_body(w_hbm, idx_hbm, out_hbm, idx_v, rows_v, sem):
    info = plsc.get_sparse_core_info()
    nw = info.num_cores * info.num_subcores
    wid = lax.axis_index("s") * info.num_cores + lax.axis_index("c")
    bpw = _N // nw
    for cc in range(bpw // _GC):
        base = wid * bpw + cc * _GC
        pltpu.sync_copy(idx_hbm.at[pl.ds(base, _GC)], idx_v)
        pltpu.async_copy(w_hbm.at[idx_v], rows_v, sem).wait()
        pltpu.sync_copy(rows_v, out_hbm.at[pl.ds(base, _GC)])


def kernel(x, W):
    x_p = jnp.transpose(x, (0, 2, 3, 1))
    z_flat = x_p.reshape(-1, _D)
    z2 = jnp.sum(z_flat ** 2, axis=1, keepdims=True)
    w2 = jnp.sum(W ** 2, axis=1).reshape(_K, 1)
    w2d = W + W

    x3 = x.reshape(8, _D, 1024)
    blocks_per_b = 1024 // _BN

    idx3, dsum = pl.pallas_call(
        _dist_argmin_body,
        grid=(_NBLK,),
        in_specs=[
            pl.BlockSpec((1, _D, _BN),
                         lambda i: (i // blocks_per_b, 0, i % blocks_per_b)),
            pl.BlockSpec((1, _BN), lambda i: (0, i)),
            pl.BlockSpec((_K, _D), lambda i: (0, 0)),
            pl.BlockSpec((_K, 1), lambda i: (0, 0)),
        ],
        out_specs=[
            pl.BlockSpec((1, 1, _BN), lambda i: (i, 0, 0)),
            pl.BlockSpec((1, 8, 128), lambda i: (i, 0, 0)),
        ],
        out_shape=[
            jax.ShapeDtypeStruct((_NBLK, 1, _BN), jnp.int32),
            jax.ShapeDtypeStruct((_NBLK, 8, 128), jnp.float32),
        ],
        compiler_params=pltpu.CompilerParams(
            dimension_semantics=("parallel",)),
    )(x3, z2.reshape(1, _N), w2d, w2)

    encoding_index = idx3.reshape(_N)

    w_pad = jnp.concatenate([W, jnp.zeros((_K, 128 - _D), jnp.float32)], axis=1)
    gather = pl.kernel(
        _gather_body,
        mesh=plsc.VectorSubcoreMesh(core_axis_name="c", subcore_axis_name="s"),
        out_type=jax.ShapeDtypeStruct((_N, 128), jnp.float32),
        scratch_types=[
            pltpu.VMEM((_GC,), jnp.int32),
            pltpu.VMEM((_GC, 128), jnp.float32),
            pltpu.SemaphoreType.DMA,
        ],
    )
    embed_flat = gather(w_pad, encoding_index)[:, :_D]

    loss = 1.25 * jnp.sum(dsum[:, 0, 0]) / (_N * _D)
    embed_out = jnp.transpose(embed_flat.reshape(8, 32, 32, _D), (0, 3, 1, 2))
    return (embed_out, loss, encoding_index)


# speedup vs baseline: 1.2756x; 1.1436x over previous
import jax
import jax.numpy as jnp
from jax import lax
from jax.experimental import pallas as pl
from jax.experimental.pallas import tpu as pltpu
from jax.experimental.pallas import tpu_sc as plsc

_K = 8192
_D = 64
_N = 8192
_BN = 512
_BK = 4096
_NBLK = _N // _BN


def _dist_argmin_body(zt_ref, z2_ref, w_ref, idx_ref, dsum_ref):
    zt = zt_ref[0]
    z2 = z2_ref[...]
    best_val = jnp.full((1, _BN), jnp.inf, jnp.float32)
    best_idx = jnp.zeros((1, _BN), jnp.int32)
    dmin = jnp.full((1, _BN), jnp.inf, jnp.float32)
    for kc in range(_K // _BK):
        w = w_ref[kc * _BK:(kc + 1) * _BK, :]
        wblk = w + w
        w2blk = jnp.sum(w * w, axis=1, keepdims=True)
        mm2 = lax.dot_general(wblk, zt, (((1,), (0,)), ((), ())),
                              preferred_element_type=jnp.float32)
        d = (z2 + w2blk) - mm2
        bmin = jnp.min(d, axis=0, keepdims=True)
        iot = lax.broadcasted_iota(jnp.int32, (_BK, _BN), 0)
        bidx = jnp.min(jnp.where(d == bmin, iot, _K),
                       axis=0, keepdims=True) + kc * _BK
        upd = bmin < best_val
        bmin16 = bmin.astype(jnp.bfloat16).astype(jnp.float32)
        best_val = jnp.where(upd, bmin16, best_val)
        best_idx = jnp.where(upd, bidx, best_idx)
        dmin = jnp.minimum(dmin, bmin)
    idx_ref[...] = best_idx.reshape(1, 1, _BN)
    dsum_ref[...] = jnp.full((1, 8, 128), jnp.sum(dmin), jnp.float32)


_GC = 128


def _gather_body(w_hbm, idx_hbm, out_hbm, idx_v, rows_v, sem):
    info = plsc.get_sparse_core_info()
    nw = info.num_cores * info.num_subcores
    wid = lax.axis_index("s") * info.num_cores + lax.axis_index("c")
    bpw = _N // nw
    for cc in range(bpw // _GC):
        base = wid * bpw + cc * _GC
        pltpu.sync_copy(idx_hbm.at[pl.ds(base, _GC)], idx_v)
        pltpu.async_copy(w_hbm.at[idx_v], rows_v, sem).wait()
        pltpu.sync_copy(rows_v, out_hbm.at[pl.ds(base, _GC)])


def kernel(x, W):
    x_p = jnp.transpose(x, (0, 2, 3, 1))
    z_flat = x_p.reshape(-1, _D)
    z2 = jnp.sum(z_flat ** 2, axis=1, keepdims=True)

    x3 = x.reshape(8, _D, 1024)
    blocks_per_b = 1024 // _BN

    idx3, dsum = pl.pallas_call(
        _dist_argmin_body,
        grid=(_NBLK,),
        in_specs=[
            pl.BlockSpec((1, _D, _BN),
                         lambda i: (i // blocks_per_b, 0, i % blocks_per_b)),
            pl.BlockSpec((1, _BN), lambda i: (0, i)),
            pl.BlockSpec((_K, _D), lambda i: (0, 0)),
        ],
        out_specs=[
            pl.BlockSpec((1, 1, _BN), lambda i: (i, 0, 0)),
            pl.BlockSpec((1, 8, 128), lambda i: (i, 0, 0)),
        ],
        out_shape=[
            jax.ShapeDtypeStruct((_NBLK, 1, _BN), jnp.int32),
            jax.ShapeDtypeStruct((_NBLK, 8, 128), jnp.float32),
        ],
        compiler_params=pltpu.CompilerParams(
            dimension_semantics=("parallel",)),
    )(x3, z2.reshape(1, _N), W)

    encoding_index = idx3.reshape(_N)

    w_pad = jnp.concatenate([W, jnp.zeros((_K, 128 - _D), jnp.float32)], axis=1)
    gather = pl.kernel(
        _gather_body,
        mesh=plsc.VectorSubcoreMesh(core_axis_name="c", subcore_axis_name="s"),
        out_type=jax.ShapeDtypeStruct((_N, 128), jnp.float32),
        scratch_types=[
            pltpu.VMEM((_GC,), jnp.int32),
            pltpu.VMEM((_GC, 128), jnp.float32),
            pltpu.SemaphoreType.DMA,
        ],
    )
    embed_flat = gather(w_pad, encoding_index)[:, :_D]

    loss = 1.25 * jnp.sum(dsum[:, 0, 0]) / (_N * _D)
    embed_out = jnp.transpose(embed_flat.reshape(8, 32, 32, _D), (0, 3, 1, 2))
    return (embed_out, loss, encoding_index)


# speedup vs baseline: 1.2787x; 1.0024x over previous
import jax
import jax.numpy as jnp
from jax import lax
from jax.experimental import pallas as pl
from jax.experimental.pallas import tpu as pltpu
from jax.experimental.pallas import tpu_sc as plsc

_K = 8192
_D = 64
_N = 8192
_BN = 512
_BK = 4096
_NBLK = _N // _BN


def _dist_argmin_body(zt_ref, z2_ref, w_ref, idx_ref, dsum_ref):
    zblk = zt_ref[...]
    z2 = z2_ref[...]
    best_val = jnp.full((1, _BN), jnp.inf, jnp.float32)
    best_idx = jnp.zeros((1, _BN), jnp.int32)
    dmin = jnp.full((1, _BN), jnp.inf, jnp.float32)
    for kc in range(_K // _BK):
        w = w_ref[kc * _BK:(kc + 1) * _BK, :]
        wblk = w + w
        w2blk = jnp.sum(w * w, axis=1, keepdims=True)
        mm2 = lax.dot_general(wblk, zblk, (((1,), (1,)), ((), ())),
                              preferred_element_type=jnp.float32)
        d = (z2 + w2blk) - mm2
        bmin = jnp.min(d, axis=0, keepdims=True)
        iot = lax.broadcasted_iota(jnp.int32, (_BK, _BN), 0)
        bidx = jnp.min(jnp.where(d == bmin, iot, _K),
                       axis=0, keepdims=True) + kc * _BK
        upd = bmin < best_val
        bmin16 = bmin.astype(jnp.bfloat16).astype(jnp.float32)
        best_val = jnp.where(upd, bmin16, best_val)
        best_idx = jnp.where(upd, bidx, best_idx)
        dmin = jnp.minimum(dmin, bmin)
    idx_ref[...] = best_idx.reshape(1, 1, _BN)
    dsum_ref[...] = jnp.full((1, 8, 128), jnp.sum(dmin), jnp.float32)


_GC = 128


def _gather_body(w_hbm, idx_hbm, out_hbm, idx_v, rows_v, sem):
    info = plsc.get_sparse_core_info()
    nw = info.num_cores * info.num_subcores
    wid = lax.axis_index("s") * info.num_cores + lax.axis_index("c")
    bpw = _N // nw
    for cc in range(bpw // _GC):
        base = wid * bpw + cc * _GC
        pltpu.sync_copy(idx_hbm.at[pl.ds(base, _GC)], idx_v)
        pltpu.async_copy(w_hbm.at[idx_v], rows_v, sem).wait()
        pltpu.sync_copy(rows_v, out_hbm.at[pl.ds(base, _GC)])


def kernel(x, W):
    x_p = jnp.transpose(x, (0, 2, 3, 1))
    z_flat = x_p.reshape(-1, _D)
    z2 = jnp.sum(z_flat ** 2, axis=1, keepdims=True)

    idx3, dsum = pl.pallas_call(
        _dist_argmin_body,
        grid=(_NBLK,),
        in_specs=[
            pl.BlockSpec((_BN, _D), lambda i: (i, 0)),
            pl.BlockSpec((1, _BN), lambda i: (0, i)),
            pl.BlockSpec((_K, _D), lambda i: (0, 0)),
        ],
        out_specs=[
            pl.BlockSpec((1, 1, _BN), lambda i: (i, 0, 0)),
            pl.BlockSpec((1, 8, 128), lambda i: (i, 0, 0)),
        ],
        out_shape=[
            jax.ShapeDtypeStruct((_NBLK, 1, _BN), jnp.int32),
            jax.ShapeDtypeStruct((_NBLK, 8, 128), jnp.float32),
        ],
        compiler_params=pltpu.CompilerParams(
            dimension_semantics=("parallel",)),
    )(z_flat, z2.reshape(1, _N), W)

    encoding_index = idx3.reshape(_N)

    w_pad = jnp.concatenate([W, jnp.zeros((_K, 128 - _D), jnp.float32)], axis=1)
    gather = pl.kernel(
        _gather_body,
        mesh=plsc.VectorSubcoreMesh(core_axis_name="c", subcore_axis_name="s"),
        out_type=jax.ShapeDtypeStruct((_N, 128), jnp.float32),
        scratch_types=[
            pltpu.VMEM((_GC,), jnp.int32),
            pltpu.VMEM((_GC, 128), jnp.float32),
            pltpu.SemaphoreType.DMA,
        ],
    )
    embed_flat = gather(w_pad, encoding_index)[:, :_D]

    loss = 1.25 * jnp.sum(dsum[:, 0, 0]) / (_N * _D)
    embed_out = jnp.transpose(embed_flat.reshape(8, 32, 32, _D), (0, 3, 1, 2))
    return (embed_out, loss, encoding_index)
